# Initial kernel scaffold; baseline (speedup 1.0000x reference)
import functools

import jax
import jax.numpy as jnp
from jax import lax
from jax.experimental import pallas as pl
from jax.experimental.pallas import tpu as pltpu
from jax.experimental.pallas import tpu_sc as plsc

N = 10000
E = 320000
NC = 2
NS = 16
NW = NC * NS
EPT = E // NW
K = 80
NCHUNK = EPT // K
NP = 10240
SP = NP // NS
DEGW = 16

_mesh = plsc.VectorSubcoreMesh(core_axis_name="c", subcore_axis_name="s")


def _zero_rows(buf, nrows, width):
    zeros = jnp.zeros((16,), jnp.float32)

    def body(i, _):
        for j in range(width // 16):
            buf[0, i, pl.ds(j * 16, 16)] = zeros
        return 0

    lax.fori_loop(0, nrows, body, 0)


@functools.partial(
    pl.kernel,
    out_type=jax.ShapeDtypeStruct((NC, NP, DEGW), jnp.float32),
    mesh=_mesh,
    scratch_types=[
        pltpu.VMEM_SHARED((NP, DEGW), jnp.float32),
        pltpu.VMEM((1, K, DEGW), jnp.float32),
        pltpu.VMEM((1, K), jnp.int32),
    ],
)
def _deg_kernel(dst_hbm, out_hbm, acc, ones, idx):
    cid = lax.axis_index("c")
    sid = lax.axis_index("s")
    wid = cid * NS + sid

    _zero_rows(ones, K, DEGW)
    for t in range(SP // K):
        pltpu.sync_copy(ones.at[0], acc.at[pl.ds(sid * SP + t * K, K)])
    ones_v = jnp.ones((16,), jnp.float32)

    def fill(i, _):
        ones[0, i, pl.ds(0, 16)] = ones_v
        return 0

    lax.fori_loop(0, K, fill, 0)
    plsc.subcore_barrier()

    def body(c, _):
        base = wid * EPT + c * K
        pltpu.sync_copy(dst_hbm.at[pl.ds(base, K)], idx.at[0])
        pltpu.sync_copy(ones.at[0], acc.at[idx.at[0]], add=True)
        return 0

    lax.fori_loop(0, NCHUNK, body, 0)
    plsc.subcore_barrier()
    pltpu.sync_copy(
        acc.at[pl.ds(sid * SP, SP)], out_hbm.at[cid, pl.ds(sid * SP, SP)]
    )


def _make_agg(d):
    @functools.partial(
        pl.kernel,
        out_type=jax.ShapeDtypeStruct((NC, NP, d), jnp.float32),
        mesh=_mesh,
        scratch_types=[
            pltpu.VMEM_SHARED((NP, d), jnp.float32),
            pltpu.VMEM((2, K, d), jnp.float32),
            pltpu.VMEM((2, K), jnp.int32),
            pltpu.VMEM((2, K), jnp.int32),
            pltpu.SemaphoreType.DMA,
        ],
    )
    def _agg(h_hbm, src_hbm, dst_hbm, out_hbm, acc, rows, sidx, didx, sem):
        cid = lax.axis_index("c")
        sid = lax.axis_index("s")
        wid = cid * NS + sid

        _zero_rows(rows, K, d)
        for t in range(SP // K):
            pltpu.sync_copy(rows.at[0], acc.at[pl.ds(sid * SP + t * K, K)])
        plsc.subcore_barrier()

        def body(c, _):
            base = wid * EPT + c * K
            pltpu.sync_copy(src_hbm.at[pl.ds(base, K)], sidx.at[0])
            pltpu.sync_copy(dst_hbm.at[pl.ds(base, K)], didx.at[0])
            pltpu.async_copy(h_hbm.at[sidx.at[0]], rows.at[0], sem).wait()
            pltpu.sync_copy(rows.at[0], acc.at[didx.at[0]], add=True)
            return 0

        lax.fori_loop(0, NCHUNK, body, 0)
        plsc.subcore_barrier()
        pltpu.sync_copy(
            acc.at[pl.ds(sid * SP, SP)], out_hbm.at[cid, pl.ds(sid * SP, SP)]
        )

    return _agg


_agg128 = _make_agg(128)
_agg64 = _make_agg(64)


BM = 1000


def _dinv_block(deg0, deg1):
    deg = deg0[:, 0:1] + deg1[:, 0:1] + 1.0
    return lax.rsqrt(jnp.maximum(deg, 1e-12))


def _mm_scale_body(x_ref, w_ref, d0_ref, d1_ref, out_ref):
    dinv = _dinv_block(d0_ref[...], d1_ref[...])
    out_ref[...] = jnp.dot(
        x_ref[...], w_ref[...], preferred_element_type=jnp.float32
    ) * dinv


def _mid_body(p0_ref, p1_ref, hp_ref, d0_ref, d1_ref, b1_ref, w2_ref, out_ref):
    dinv = _dinv_block(d0_ref[...], d1_ref[...])
    a = dinv * (p0_ref[...] + p1_ref[...] + hp_ref[...]) + b1_ref[...]
    z = jnp.maximum(a, 0.0)
    out_ref[...] = jnp.dot(
        z, w2_ref[...], preferred_element_type=jnp.float32
    ) * dinv


def _final_body(q0_ref, q1_ref, hp_ref, d0_ref, d1_ref, b2_ref, out_ref):
    dinv = _dinv_block(d0_ref[...], d1_ref[...])
    s = dinv * (q0_ref[...] + q1_ref[...] + hp_ref[...]) + b2_ref[...]
    m = jnp.max(s, axis=1, keepdims=True)
    e = jnp.exp(s - m)
    out_ref[...] = e / jnp.sum(e, axis=1, keepdims=True)


def _row_spec(w):
    return pl.BlockSpec((BM, w), lambda i: (i, 0))


def kernel(x, edge_index, W1, b1, W2, b2):
    ei = edge_index.astype(jnp.int32)
    src = ei[0]
    dst = ei[1]

    deg_parts = _deg_kernel(dst)
    deg0 = deg_parts[0, :N]
    deg1 = deg_parts[1, :N]

    grid = (N // BM,)

    h1p = pl.pallas_call(
        _mm_scale_body,
        grid=grid,
        in_specs=[
            _row_spec(128),
            pl.BlockSpec((128, 128), lambda i: (0, 0)),
            _row_spec(DEGW),
            _row_spec(DEGW),
        ],
        out_specs=_row_spec(128),
        out_shape=jax.ShapeDtypeStruct((N, 128), jnp.float32),
    )(x, W1, deg0, deg1)

    p = _agg128(h1p, src, dst)

    h2p = pl.pallas_call(
        _mid_body,
        grid=grid,
        in_specs=[
            _row_spec(128),
            _row_spec(128),
            _row_spec(128),
            _row_spec(DEGW),
            _row_spec(DEGW),
            pl.BlockSpec((128,), lambda i: (0,)),
            pl.BlockSpec((128, 64), lambda i: (0, 0)),
        ],
        out_specs=_row_spec(64),
        out_shape=jax.ShapeDtypeStruct((N, 64), jnp.float32),
    )(p[0, :N], p[1, :N], h1p, deg0, deg1, b1, W2)

    q = _agg64(h2p, src, dst)

    out = pl.pallas_call(
        _final_body,
        grid=grid,
        in_specs=[
            _row_spec(64),
            _row_spec(64),
            _row_spec(64),
            _row_spec(DEGW),
            _row_spec(DEGW),
            pl.BlockSpec((64,), lambda i: (0,)),
        ],
        out_specs=_row_spec(64),
        out_shape=jax.ShapeDtypeStruct((N, 64), jnp.float32),
    )(q[0, :N], q[1, :N], h2p, deg0, deg1, b2)

    return out


# speedup vs baseline: 13.4273x; 13.4273x over previous
import functools

import jax
import jax.numpy as jnp
from jax import lax
from jax.experimental import pallas as pl
from jax.experimental.pallas import tpu as pltpu
from jax.experimental.pallas import tpu_sc as plsc

N = 10000
E = 320000
NC = 2
NS = 16
NW = NC * NS
EPT = E // NW
K = 80
NCHUNK = EPT // K
NP = 10240
SP = NP // NS
DEGW = 16

_mesh = plsc.VectorSubcoreMesh(
    core_axis_name="c", subcore_axis_name="s", num_cores=NC, num_subcores=NS
)


def _zero_rows(buf, nrows, width):
    zeros = jnp.zeros((16,), jnp.float32)

    def body(i, _):
        for j in range(width // 16):
            buf[0, i, pl.ds(j * 16, 16)] = zeros
        return 0

    lax.fori_loop(0, nrows, body, 0)


@functools.partial(
    pl.kernel,
    out_type=jax.ShapeDtypeStruct((NC, NP, DEGW), jnp.float32),
    mesh=_mesh,
    scratch_types=[
        pltpu.VMEM_SHARED((NP, DEGW), jnp.float32),
        pltpu.VMEM((1, K, DEGW), jnp.float32),
        pltpu.VMEM((1, K), jnp.int32),
    ],
    compiler_params=pltpu.CompilerParams(use_tc_tiling_on_sc=False),
)
def _deg_kernel(dst_hbm, out_hbm, acc, ones, idx):
    cid = lax.axis_index("c")
    sid = lax.axis_index("s")
    wid = cid * NS + sid

    _zero_rows(ones, K, DEGW)
    for t in range(SP // K):
        pltpu.sync_copy(ones.at[0], acc.at[pl.ds(sid * SP + t * K, K)])
    ones_v = jnp.ones((16,), jnp.float32)

    def fill(i, _):
        ones[0, i, pl.ds(0, 16)] = ones_v
        return 0

    lax.fori_loop(0, K, fill, 0)
    plsc.subcore_barrier()

    def body(c, _):
        base = wid * EPT + c * K
        pltpu.sync_copy(dst_hbm.at[pl.ds(base, K)], idx.at[0])
        pltpu.sync_copy(ones.at[0], acc.at[idx.at[0]], add=True)
        return 0

    lax.fori_loop(0, NCHUNK, body, 0)
    plsc.subcore_barrier()
    pltpu.sync_copy(
        acc.at[pl.ds(sid * SP, SP)], out_hbm.at[cid, pl.ds(sid * SP, SP)]
    )


def _make_agg(d):
    @functools.partial(
        pl.kernel,
        out_type=jax.ShapeDtypeStruct((NC, NP, d), jnp.float32),
        mesh=_mesh,
        scratch_types=[
            pltpu.VMEM_SHARED((NP, d), jnp.float32),
            pltpu.VMEM((2, K, d), jnp.float32),
            pltpu.VMEM((2, K), jnp.int32),
            pltpu.VMEM((2, K), jnp.int32),
            pltpu.SemaphoreType.DMA,
        ],
        compiler_params=pltpu.CompilerParams(use_tc_tiling_on_sc=False),
    )
    def _agg(h_hbm, src_hbm, dst_hbm, out_hbm, acc, rows, sidx, didx, sem):
        cid = lax.axis_index("c")
        sid = lax.axis_index("s")
        wid = cid * NS + sid

        _zero_rows(rows, K, d)
        for t in range(SP // K):
            pltpu.sync_copy(rows.at[0], acc.at[pl.ds(sid * SP + t * K, K)])
        plsc.subcore_barrier()

        def body(c, _):
            base = wid * EPT + c * K
            pltpu.sync_copy(src_hbm.at[pl.ds(base, K)], sidx.at[0])
            pltpu.sync_copy(dst_hbm.at[pl.ds(base, K)], didx.at[0])
            pltpu.async_copy(h_hbm.at[sidx.at[0]], rows.at[0], sem).wait()
            pltpu.sync_copy(rows.at[0], acc.at[didx.at[0]], add=True)
            return 0

        lax.fori_loop(0, NCHUNK, body, 0)
        plsc.subcore_barrier()
        pltpu.sync_copy(
            acc.at[pl.ds(sid * SP, SP)], out_hbm.at[cid, pl.ds(sid * SP, SP)]
        )

    return _agg


_agg128 = _make_agg(128)
_agg64 = _make_agg(64)


BM = 1000


def _dinv_block(deg0, deg1):
    deg = deg0[:, 0:1] + deg1[:, 0:1] + 1.0
    return lax.rsqrt(jnp.maximum(deg, 1e-12))


def _mm_scale_body(x_ref, w_ref, d0_ref, d1_ref, out_ref):
    dinv = _dinv_block(d0_ref[...], d1_ref[...])
    out_ref[...] = jnp.dot(
        x_ref[...], w_ref[...], preferred_element_type=jnp.float32
    ) * dinv


def _mid_body(p0_ref, p1_ref, hp_ref, d0_ref, d1_ref, b1_ref, w2_ref, out_ref):
    dinv = _dinv_block(d0_ref[...], d1_ref[...])
    a = dinv * (p0_ref[...] + p1_ref[...] + hp_ref[...]) + b1_ref[...]
    z = jnp.maximum(a, 0.0)
    out_ref[...] = jnp.dot(
        z, w2_ref[...], preferred_element_type=jnp.float32
    ) * dinv


def _final_body(q0_ref, q1_ref, hp_ref, d0_ref, d1_ref, b2_ref, out_ref):
    dinv = _dinv_block(d0_ref[...], d1_ref[...])
    s = dinv * (q0_ref[...] + q1_ref[...] + hp_ref[...]) + b2_ref[...]
    m = jnp.max(s, axis=1, keepdims=True)
    e = jnp.exp(s - m)
    out_ref[...] = e / jnp.sum(e, axis=1, keepdims=True)


def _row_spec(w):
    return pl.BlockSpec((BM, w), lambda i: (i, 0))


def kernel(x, edge_index, W1, b1, W2, b2):
    ei = edge_index.astype(jnp.int32)
    src = ei[0]
    dst = ei[1]

    deg_parts = _deg_kernel(dst)
    deg0 = deg_parts[0, :N]
    deg1 = deg_parts[1, :N]

    grid = (N // BM,)

    h1p = pl.pallas_call(
        _mm_scale_body,
        grid=grid,
        in_specs=[
            _row_spec(128),
            pl.BlockSpec((128, 128), lambda i: (0, 0)),
            _row_spec(DEGW),
            _row_spec(DEGW),
        ],
        out_specs=_row_spec(128),
        out_shape=jax.ShapeDtypeStruct((N, 128), jnp.float32),
    )(x, W1, deg0, deg1)

    p = _agg128(h1p, src, dst)

    h2p = pl.pallas_call(
        _mid_body,
        grid=grid,
        in_specs=[
            _row_spec(128),
            _row_spec(128),
            _row_spec(128),
            _row_spec(DEGW),
            _row_spec(DEGW),
            pl.BlockSpec((128,), lambda i: (0,)),
            pl.BlockSpec((128, 64), lambda i: (0, 0)),
        ],
        out_specs=_row_spec(64),
        out_shape=jax.ShapeDtypeStruct((N, 64), jnp.float32),
    )(p[0, :N], p[1, :N], h1p, deg0, deg1, b1, W2)

    q = _agg64(h2p, src, dst)

    out = pl.pallas_call(
        _final_body,
        grid=grid,
        in_specs=[
            _row_spec(64),
            _row_spec(64),
            _row_spec(64),
            _row_spec(DEGW),
            _row_spec(DEGW),
            pl.BlockSpec((64,), lambda i: (0,)),
        ],
        out_specs=_row_spec(64),
        out_shape=jax.ShapeDtypeStruct((N, 64), jnp.float32),
    )(q[0, :N], q[1, :N], h2p, deg0, deg1, b2)

    return out


# speedup vs baseline: 32.4740x; 2.4185x over previous
import functools

import jax
import jax.numpy as jnp
from jax import lax
from jax.experimental import pallas as pl
from jax.experimental.pallas import tpu as pltpu
from jax.experimental.pallas import tpu_sc as plsc

N = 10000
E = 320000
NC = 2
NS = 16
NW = NC * NS
EPT = E // NW
K = 80
NCHUNK = EPT // K
SP = N // NS

_mesh = plsc.VectorSubcoreMesh(
    core_axis_name="c", subcore_axis_name="s", num_cores=NC, num_subcores=NS
)
_params = pltpu.CompilerParams(use_tc_tiling_on_sc=False, needs_layout_passes=False)


def _fill_rows(buf, nrows, width, value):
    v = jnp.full((16,), value, jnp.float32)

    def body(i, _):
        for j in range(width // 16):
            buf[i, pl.ds(j * 16, 16)] = v
        return 0

    lax.fori_loop(0, nrows, body, 0)


@functools.partial(
    pl.kernel,
    out_type=jax.ShapeDtypeStruct((NW, N), jnp.float32),
    mesh=_mesh,
    scratch_types=[
        pltpu.VMEM((N,), jnp.float32),
        pltpu.VMEM((EPT,), jnp.int32),
    ],
    compiler_params=_params,
)
def _deg_kernel(dst_hbm, out_hbm, acc, idx):
    cid = lax.axis_index("c")
    sid = lax.axis_index("s")
    wid = cid * NS + sid

    pltpu.sync_copy(dst_hbm.at[pl.ds(wid * EPT, EPT)], idx)
    zeros = jnp.zeros((16,), jnp.float32)

    def zbody(i, _):
        acc[pl.ds(i * 16, 16)] = zeros
        return 0

    lax.fori_loop(0, N // 16, zbody, 0)
    ones = jnp.ones((16,), jnp.float32)

    def body(i, _):
        iv = idx[pl.ds(i * 16, 16)]
        plsc.addupdate_scatter(acc, [iv], ones)
        return 0

    lax.fori_loop(0, EPT // 16, body, 0)
    pltpu.sync_copy(acc, out_hbm.at[wid])


def _make_agg(d):
    @functools.partial(
        pl.kernel,
        out_type=jax.ShapeDtypeStruct((NC, N, d), jnp.float32),
        mesh=_mesh,
        scratch_types=[
            pltpu.VMEM_SHARED((N, d), jnp.float32),
            pltpu.VMEM((2, K, d), jnp.float32),
            pltpu.VMEM((NCHUNK, K), jnp.int32),
            pltpu.VMEM((NCHUNK, K), jnp.int32),
            pltpu.SemaphoreType.DMA,
            pltpu.SemaphoreType.DMA,
        ],
        compiler_params=_params,
    )
    def _agg(h_hbm, src_hbm, dst_hbm, out_hbm,
             acc, rows, sidx, didx, sem0, sem1):
        cid = lax.axis_index("c")
        sid = lax.axis_index("s")
        wid = cid * NS + sid
        sems = (sem0, sem1)

        pltpu.sync_copy(src_hbm.at[pl.ds(wid * NCHUNK, NCHUNK)], sidx)
        pltpu.sync_copy(dst_hbm.at[pl.ds(wid * NCHUNK, NCHUNK)], didx)
        _fill_rows(rows.at[0], K, d, 0.0)
        for t in range(SP // K):
            pltpu.sync_copy(rows.at[0], acc.at[pl.ds(sid * SP + t * K, K)])
        rem = SP - (SP // K) * K
        pltpu.sync_copy(
            rows.at[0, pl.ds(0, rem)],
            acc.at[pl.ds(sid * SP + (SP // K) * K, rem)],
        )
        plsc.subcore_barrier()

        def gather(c, b):
            pltpu.async_copy(h_hbm.at[sidx.at[c]], rows.at[b], sems[b])

        def wait_scatter(c, b):
            pltpu.make_async_copy(
                h_hbm.at[sidx.at[c]], rows.at[b], sems[b]
            ).wait()
            pltpu.sync_copy(rows.at[b], acc.at[didx.at[c]], add=True)

        gather(0, 0)
        gather(1, 1)

        def body(i, _):
            c = 2 * i
            wait_scatter(c, 0)
            gather(c + 2, 0)
            wait_scatter(c + 1, 1)
            gather(c + 3, 1)
            return 0

        lax.fori_loop(0, NCHUNK // 2 - 1, body, 0)
        wait_scatter(NCHUNK - 3, 0)
        gather(NCHUNK - 1, 0)
        wait_scatter(NCHUNK - 2, 1)
        wait_scatter(NCHUNK - 1, 0)

        plsc.subcore_barrier()
        pltpu.sync_copy(
            acc.at[pl.ds(sid * SP, SP)], out_hbm.at[cid, pl.ds(sid * SP, SP)]
        )

    return _agg


_agg128 = _make_agg(128)
_agg64 = _make_agg(64)


BM = 1000


def _mm_body(x_ref, w_ref, out_ref):
    out_ref[...] = jnp.dot(
        x_ref[...], w_ref[...], preferred_element_type=jnp.float32
    )


def _scale_body(dp_ref, h_ref, dinv_ref, hp_ref):
    deg = jnp.sum(dp_ref[...], axis=0)[:, None] + 1.0
    dinv = lax.rsqrt(jnp.maximum(deg, 1e-12))
    dinv_ref[...] = dinv
    hp_ref[...] = h_ref[...] * dinv


def _mid_body(p0_ref, p1_ref, hp_ref, dinv_ref, b1_ref, w2_ref, out_ref):
    dinv = dinv_ref[...]
    a = dinv * (p0_ref[...] + p1_ref[...] + hp_ref[...]) + b1_ref[...]
    z = jnp.maximum(a, 0.0)
    out_ref[...] = jnp.dot(
        z, w2_ref[...], preferred_element_type=jnp.float32
    ) * dinv


def _final_body(q0_ref, q1_ref, hp_ref, dinv_ref, b2_ref, out_ref):
    dinv = dinv_ref[...]
    s = dinv * (q0_ref[...] + q1_ref[...] + hp_ref[...]) + b2_ref[...]
    m = jnp.max(s, axis=1, keepdims=True)
    e = jnp.exp(s - m)
    out_ref[...] = e / jnp.sum(e, axis=1, keepdims=True)


def _row_spec(w):
    return pl.BlockSpec((BM, w), lambda i: (i, 0))


def _dinv_spec():
    return pl.BlockSpec((BM, 1), lambda i: (i, 0))


def kernel(x, edge_index, W1, b1, W2, b2):
    ei = edge_index.astype(jnp.int32)
    src2 = ei[0].reshape(NW * NCHUNK, K)
    dst2 = ei[1].reshape(NW * NCHUNK, K)

    deg_parts = _deg_kernel(ei[1])

    grid = (N // BM,)

    h1 = pl.pallas_call(
        _mm_body,
        grid=grid,
        in_specs=[
            _row_spec(128),
            pl.BlockSpec((128, 128), lambda i: (0, 0)),
        ],
        out_specs=_row_spec(128),
        out_shape=jax.ShapeDtypeStruct((N, 128), jnp.float32),
    )(x, W1)

    dinv, h1p = pl.pallas_call(
        _scale_body,
        out_shape=[
            jax.ShapeDtypeStruct((N, 1), jnp.float32),
            jax.ShapeDtypeStruct((N, 128), jnp.float32),
        ],
    )(deg_parts, h1)

    p = _agg128(h1p, src2, dst2)

    h2p = pl.pallas_call(
        _mid_body,
        grid=grid,
        in_specs=[
            _row_spec(128),
            _row_spec(128),
            _row_spec(128),
            _dinv_spec(),
            pl.BlockSpec((128,), lambda i: (0,)),
            pl.BlockSpec((128, 64), lambda i: (0, 0)),
        ],
        out_specs=_row_spec(64),
        out_shape=jax.ShapeDtypeStruct((N, 64), jnp.float32),
    )(p[0], p[1], h1p, dinv, b1, W2)

    q = _agg64(h2p, src2, dst2)

    out = pl.pallas_call(
        _final_body,
        grid=grid,
        in_specs=[
            _row_spec(64),
            _row_spec(64),
            _row_spec(64),
            _dinv_spec(),
            pl.BlockSpec((64,), lambda i: (0,)),
        ],
        out_specs=_row_spec(64),
        out_shape=jax.ShapeDtypeStruct((N, 64), jnp.float32),
    )(q[0], q[1], h2p, dinv, b2)

    return out
